# baseline (speedup 1.0000x reference)
import jax
import jax.numpy as jnp
from jax.experimental import pallas as pl

_DEPTH = 1000
_ROWS = 16384
_RBLK = 4096


def _onehot_body(x_ref, o_ref):
    idx = x_ref[...]
    iota = jax.lax.broadcasted_iota(jnp.int32, (_DEPTH, _RBLK), 0)
    o_ref[...] = (iota == idx).astype(jnp.float32)


def kernel(x):
    xi = x.astype(jnp.int32).reshape(1, _ROWS)
    out = pl.pallas_call(
        _onehot_body,
        grid=(_ROWS // _RBLK,),
        in_specs=[pl.BlockSpec((1, _RBLK), lambda i: (0, i))],
        out_specs=pl.BlockSpec((_DEPTH, _RBLK), lambda i: (0, i)),
        out_shape=jax.ShapeDtypeStruct((_DEPTH, _ROWS), jnp.float32),
    )(xi)
    return out.T.reshape(_ROWS, 1, _DEPTH)


# speedup vs baseline: 1.1134x; 1.1134x over previous
import jax
import jax.numpy as jnp
from jax.experimental import pallas as pl

_DEPTH = 1000
_ROWS = 16384
_RBLK = 1024


def _onehot_body(x_ref, o_ref):
    idx = x_ref[...]
    iota = jax.lax.broadcasted_iota(jnp.int32, (_DEPTH, _RBLK), 0)
    o_ref[...] = (iota == idx).astype(jnp.float32)


def kernel(x):
    xi = x.astype(jnp.int32).reshape(1, _ROWS)
    out = pl.pallas_call(
        _onehot_body,
        grid=(_ROWS // _RBLK,),
        in_specs=[pl.BlockSpec((1, _RBLK), lambda i: (0, i))],
        out_specs=pl.BlockSpec((_DEPTH, _RBLK), lambda i: (0, i)),
        out_shape=jax.ShapeDtypeStruct((_DEPTH, _ROWS), jnp.float32),
    )(xi)
    return out.T.reshape(_ROWS, 1, _DEPTH)
